# Initial kernel scaffold; baseline (speedup 1.0000x reference)
import jax
import jax.numpy as jnp
from jax.experimental import pallas as pl


def kernel(x, A_r, S_r, b_r, domain_routing, A1, S1, b1, W2, b2, domain_id):
    raise NotImplementedError("write your pallas kernel here")


# speedup vs baseline: 2.2332x; 2.2332x over previous
import functools

import jax
import jax.numpy as jnp
from jax import lax
from jax.experimental import pallas as pl
from jax.experimental.pallas import tpu as pltpu
from jax.experimental.pallas import tpu_sc as plsc

T = 4096
C = 1024
E = 8
ED = 4096
RT = 1024
NT = 16
PBUF = NT * RT
NJ = 4
NW = 32

_SQRT_HALF = 0.7071067811865476


def _router_body(x_ref, wr_ref, bias_ref, rec_ref):
    xb = x_ref[...]
    wr = wr_ref[...]
    logits = lax.dot_general(xb.astype(jnp.bfloat16), wr.astype(jnp.bfloat16),
                             (((1,), (1,)), ((), ())),
                             preferred_element_type=jnp.float32)
    logits = logits + bias_ref[...]
    rows = logits.shape[0]
    iota8 = lax.broadcasted_iota(jnp.int32, (rows, E), 1)
    m1 = jnp.max(logits, axis=1, keepdims=True)
    i1 = jnp.min(jnp.where(logits >= m1, iota8, E + 1), axis=1, keepdims=True)
    l2 = jnp.where(iota8 == i1, -jnp.inf, logits)
    m2 = jnp.max(l2, axis=1, keepdims=True)
    i2 = jnp.min(jnp.where(l2 >= m2, iota8, E + 1), axis=1, keepdims=True)
    ew = jnp.exp(m2 - m1)
    w0 = 1.0 / (1.0 + ew)
    w1 = 1.0 - w0
    colid = lax.broadcasted_iota(jnp.int32, (rows, 128), 1)
    rec = jnp.where(colid == 0, i1.astype(jnp.float32), 0.0)
    rec = jnp.where(colid == 1, i2.astype(jnp.float32), rec)
    rec = jnp.where(colid == 2, w0, rec)
    rec = jnp.where(colid == 3, w1, rec)
    rec_ref[...] = rec


def _router(xf, wr, bias):
    return pl.pallas_call(
        _router_body,
        grid=(T // 512,),
        in_specs=[
            pl.BlockSpec((512, C), lambda i: (i, 0)),
            pl.BlockSpec((E, C), lambda i: (0, 0)),
            pl.BlockSpec((1, E), lambda i: (0, 0)),
        ],
        out_specs=pl.BlockSpec((512, 128), lambda i: (i, 0)),
        out_shape=jax.ShapeDtypeStruct((T, 128), jnp.float32),
    )(xf, wr, bias)


def _ffn_body(te_ref, xd_ref, a_ref, s1_ref, w2_ref, b1_ref, b2_ref, wt_ref,
              yd_ref, acc_ref):
    i = pl.program_id(0)
    j = pl.program_id(1)
    active = te_ref[NT + i]

    @pl.when(active > 0)
    def _():
        xb = xd_ref[...].astype(jnp.bfloat16)
        h = jnp.zeros((RT, C), jnp.float32)
        for n in range(2):
            s1n = s1_ref[0, n, 0].astype(jnp.bfloat16)
            for q in range(2):
                d = lax.dot_general(xb[:, q * 512:(q + 1) * 512], s1n,
                                    (((1,), (1,)), ((), ())),
                                    preferred_element_type=jnp.float32)
                h = h + a_ref[0, 0, n, q] * d
        h = h + b1_ref[0, 0]
        h = 0.5 * h * (1.0 + lax.erf(h * _SQRT_HALF))
        contrib = lax.dot_general(h.astype(jnp.bfloat16),
                                  w2_ref[0, 0].astype(jnp.bfloat16),
                                  (((1,), (0,)), ((), ())),
                                  preferred_element_type=jnp.float32)

        @pl.when(j == 0)
        def _():
            acc_ref[...] = contrib

        @pl.when(j > 0)
        def _():
            acc_ref[...] = acc_ref[...] + contrib

        @pl.when(j == NJ - 1)
        def _():
            yd_ref[...] = (acc_ref[...] + b2_ref[0]) * wt_ref[...]


def _ffn(te_pack, xd, a1p, s1r, w2r, b1r, b2r, wt):
    grid_spec = pltpu.PrefetchScalarGridSpec(
        num_scalar_prefetch=1,
        grid=(NT, NJ),
        in_specs=[
            pl.BlockSpec((RT, C), lambda i, j, te: (i, 0)),
            pl.BlockSpec((1, 1, 2, 2), lambda i, j, te: (te[i], j % 2, 0, 0)),
            pl.BlockSpec((1, 2, 1, 1024, 512),
                         lambda i, j, te: (te[i], 0, j // 2, 0, 0)),
            pl.BlockSpec((1, 1, 1024, C),
                         lambda i, j, te: (te[i], (j % 2) * 2 + j // 2, 0, 0)),
            pl.BlockSpec((1, 1, 1, ED // NJ),
                         lambda i, j, te: (te[i], (j % 2) * 2 + j // 2, 0, 0)),
            pl.BlockSpec((1, 1, C), lambda i, j, te: (te[i], 0, 0)),
            pl.BlockSpec((RT, 1), lambda i, j, te: (i, 0)),
        ],
        out_specs=pl.BlockSpec((RT, C), lambda i, j, te: (i, 0)),
        scratch_shapes=[pltpu.VMEM((RT, C), jnp.float32)],
    )
    return pl.pallas_call(
        _ffn_body,
        grid_spec=grid_spec,
        out_shape=jax.ShapeDtypeStruct((PBUF, C), jnp.float32),
        compiler_params=pltpu.CompilerParams(
            dimension_semantics=("arbitrary", "arbitrary")),
    )(te_pack, xd, a1p, s1r, w2r, b1r, b2r, wt)


def _gather_rows(xf, tok_of):
    mesh = plsc.VectorSubcoreMesh(core_axis_name="c", subcore_axis_name="s")
    rows_per_w = PBUF // NW
    chunk = 64
    nchunk = rows_per_w // chunk

    @functools.partial(
        pl.kernel,
        out_type=jax.ShapeDtypeStruct((PBUF, C), jnp.float32),
        mesh=mesh,
        scratch_types=[
            pltpu.VMEM((chunk,), jnp.int32),
            pltpu.VMEM((chunk, C), jnp.float32),
            pltpu.SemaphoreType.DMA,
        ],
    )
    def gather_k(xf_hbm, tok_hbm, out_hbm, idx_v, rows_v, sem):
        wid = lax.axis_index("s") * 2 + lax.axis_index("c")
        base = wid * rows_per_w

        def step(cidx, carry):
            off = base + cidx * chunk
            pltpu.sync_copy(tok_hbm.at[pl.ds(off, chunk)], idx_v)
            pltpu.async_copy(xf_hbm.at[idx_v], rows_v, sem).wait()
            pltpu.sync_copy(rows_v, out_hbm.at[pl.ds(off, chunk)])
            return carry

        lax.fori_loop(0, nchunk, step, 0)

    return gather_k(xf, tok_of)


def _combine(yd, p0, p1):
    mesh = plsc.VectorSubcoreMesh(core_axis_name="c", subcore_axis_name="s")
    rows_per_w = T // NW
    chunk = 32
    nchunk = rows_per_w // chunk

    @functools.partial(
        pl.kernel,
        out_type=jax.ShapeDtypeStruct((T, C), jnp.float32),
        mesh=mesh,
        scratch_types=[
            pltpu.VMEM((chunk,), jnp.int32),
            pltpu.VMEM((chunk,), jnp.int32),
            pltpu.VMEM((chunk, C), jnp.float32),
            pltpu.VMEM((chunk, C), jnp.float32),
            pltpu.SemaphoreType.DMA,
            pltpu.SemaphoreType.DMA,
        ],
    )
    def combine_k(yd_hbm, p0_hbm, p1_hbm, out_hbm, i0_v, i1_v, a_v, b_v,
                  sem0, sem1):
        wid = lax.axis_index("s") * 2 + lax.axis_index("c")
        base = wid * rows_per_w

        def step(cidx, carry):
            off = base + cidx * chunk
            pltpu.sync_copy(p0_hbm.at[pl.ds(off, chunk)], i0_v)
            pltpu.sync_copy(p1_hbm.at[pl.ds(off, chunk)], i1_v)
            cp0 = pltpu.async_copy(yd_hbm.at[i0_v], a_v, sem0)
            cp1 = pltpu.async_copy(yd_hbm.at[i1_v], b_v, sem1)
            cp0.wait()
            cp1.wait()
            for r in range(chunk):
                def add_vec(k, c):
                    sl = pl.ds(k * 16, 16)
                    a_v[r, sl] = a_v[r, sl] + b_v[r, sl]
                    return c
                lax.fori_loop(0, C // 16, add_vec, 0)
            pltpu.sync_copy(a_v, out_hbm.at[pl.ds(off, chunk)])
            return carry

        lax.fori_loop(0, nchunk, step, 0)

    return combine_k(yd, p0, p1)


def _dispatch(i1, i2, w0, w1):
    ef = jnp.stack([i1, i2], axis=1).reshape(-1)
    wf = jnp.stack([w0, w1], axis=1).reshape(-1)
    onehot = (ef[:, None] == jnp.arange(E, dtype=jnp.int32)[None, :])
    counts = jnp.sum(onehot.astype(jnp.int32), axis=0)
    padded = ((counts + RT - 1) // RT) * RT
    ends_p = jnp.cumsum(padded)
    gs = ends_p - padded
    starts = jnp.cumsum(counts) - counts
    order = jnp.argsort(ef, stable=True)
    ef_s = ef[order]
    pos_s = gs[ef_s] + (jnp.arange(2 * T, dtype=jnp.int32) - starts[ef_s])
    pos = jnp.zeros((2 * T,), jnp.int32).at[order].set(pos_s)
    tok_of = jnp.zeros((PBUF,), jnp.int32).at[pos].set(
        jnp.arange(2 * T, dtype=jnp.int32) // 2)
    wt_of = jnp.zeros((PBUF,), jnp.float32).at[pos].set(wf)
    tile_base = jnp.arange(NT, dtype=jnp.int32) * RT
    te = jnp.searchsorted(ends_p, tile_base, side="right").astype(jnp.int32)
    te = jnp.minimum(te, E - 1)
    active = (tile_base < ends_p[-1]).astype(jnp.int32)
    te_pack = jnp.concatenate([te, active])
    posr = pos.reshape(T, 2)
    return tok_of, wt_of.reshape(PBUF, 1), te_pack, posr[:, 0], posr[:, 1]


def kernel(x, A_r, S_r, b_r, domain_routing, A1, S1, b1, W2, b2, domain_id):
    Bb, N, Cc = x.shape
    xf = x.reshape(T, C)
    wr = jnp.einsum('npq,nkl->pkql', A_r, S_r).reshape(E, C)
    bias = (b_r + domain_routing[domain_id])[None, :]

    rec = _router(xf, wr, bias)
    i1 = rec[:, 0].astype(jnp.int32)
    i2 = rec[:, 1].astype(jnp.int32)
    tok_of, wt_of, te_pack, p0, p1 = _dispatch(i1, i2, rec[:, 2], rec[:, 3])

    xd = _gather_rows(xf, tok_of)

    a1p = A1.transpose(0, 2, 1, 3)
    s1r = S1.reshape(E, 2, 2, 1024, 512)
    w2r = W2.reshape(E, NJ, ED // NJ, C)
    b1r = b1.reshape(E, NJ, 1, ED // NJ)
    b2r = b2.reshape(E, 1, C)
    yd = _ffn(te_pack, xd, a1p, s1r, w2r, b1r, b2r, wt_of)

    out = _combine(yd, p0, p1)
    return out.reshape(Bb, N, Cc)
